# baseline (speedup 1.0000x reference)
import functools

import jax
import jax.numpy as jnp
from jax import lax
from jax.experimental import pallas as pl
from jax.experimental.pallas import tpu as pltpu
from jax.experimental.pallas import tpu_sc as plsc

B = 1024
FEAT = 3 * 64 * 64
S_ROWS = 4096
CHUNK = 16
RB = 2048


@functools.lru_cache(maxsize=1)
def _make_sc_gather():
    info = plsc.get_sparse_core_info()
    nc, ns = info.num_cores, info.num_subcores
    nw = nc * ns
    bpw = B // nw

    mesh = plsc.VectorSubcoreMesh(core_axis_name="c", subcore_axis_name="s")

    @functools.partial(
        pl.kernel,
        mesh=mesh,
        out_type=[
            jax.ShapeDtypeStruct((B,), jnp.float32),
            jax.ShapeDtypeStruct((B,), jnp.float32),
        ],
        scratch_types=[
            pltpu.VMEM((bpw,), jnp.int32),
            pltpu.VMEM((bpw,), jnp.float32),
            pltpu.VMEM((bpw,), jnp.float32),
            pltpu.SemaphoreType.DMA,
        ],
    )
    def sc_gather(t_hbm, sa_hbm, sb_hbm, out_a_hbm, out_b_hbm, idx_v, a_v, b_v, sem):
        wid = lax.axis_index("s") * nc + lax.axis_index("c")
        base = wid * bpw
        pltpu.sync_copy(t_hbm.at[pl.ds(base, bpw)], idx_v)
        ca = pltpu.async_copy(sa_hbm.at[idx_v], a_v, sem)
        cb = pltpu.async_copy(sb_hbm.at[idx_v], b_v, sem)
        ca.wait()
        cb.wait()
        pltpu.sync_copy(a_v, out_a_hbm.at[pl.ds(base, bpw)])
        pltpu.sync_copy(b_v, out_b_hbm.at[pl.ds(base, bpw)])

    return sc_gather


@functools.lru_cache(maxsize=1)
def _make_sc_elem():
    info = plsc.get_sparse_core_info()
    nc, ns = info.num_cores, info.num_subcores
    nw = nc * ns
    rpw = S_ROWS // nw
    nch = rpw // CHUNK

    mesh = plsc.VectorSubcoreMesh(core_axis_name="c", subcore_axis_name="s")

    @functools.partial(
        pl.kernel,
        mesh=mesh,
        out_type=jax.ShapeDtypeStruct((S_ROWS, B), jnp.float32),
        scratch_types=[
            pltpu.VMEM((2, CHUNK, B), jnp.float32),
            pltpu.VMEM((2, CHUNK, B), jnp.float32),
            pltpu.VMEM((2, CHUNK, B), jnp.float32),
            pltpu.VMEM((B,), jnp.float32),
            pltpu.VMEM((B,), jnp.float32),
            pltpu.SemaphoreType.DMA,
            pltpu.SemaphoreType.DMA,
            pltpu.SemaphoreType.DMA,
            pltpu.SemaphoreType.DMA,
        ],
    )
    def sc_elem(x_hbm, e_hbm, sa_hbm, sb_hbm, out_hbm,
                xb, eb, ob, sab, sbb, is0, is1, os0, os1):
        wid = lax.axis_index("s") * nc + lax.axis_index("c")
        row0 = wid * rpw
        pltpu.sync_copy(sa_hbm, sab)
        pltpu.sync_copy(sb_hbm, sbb)
        isems = (is0, is1)
        osems = (os0, os1)

        def issue_in(k, buf):
            r = row0 + k * CHUNK
            cx = pltpu.async_copy(x_hbm.at[pl.ds(r, CHUNK)], xb.at[buf], isems[buf])
            ce = pltpu.async_copy(e_hbm.at[pl.ds(r, CHUNK)], eb.at[buf], isems[buf])
            return cx, ce

        def compute(buf):
            def jbody(j, _):
                sl = pl.ds(j * 16, 16)
                sa_v = sab[sl]
                sb_v = sbb[sl]
                for r in range(CHUNK):
                    ob[buf, r, sl] = xb[buf, r, sl] * sa_v + eb[buf, r, sl] * sb_v
                return 0
            lax.fori_loop(0, B // 16, jbody, 0)

        pending_in = {0: issue_in(0, 0)}
        pending_out = {}
        for k in range(nch):
            buf = k % 2
            cx, ce = pending_in.pop(k)
            cx.wait()
            ce.wait()
            if k + 1 < nch:
                pending_in[k + 1] = issue_in(k + 1, 1 - buf)
            if k >= 2:
                pending_out.pop(k - 2).wait()
            compute(buf)
            r = row0 + k * CHUNK
            pending_out[k] = pltpu.async_copy(
                ob.at[buf], out_hbm.at[pl.ds(r, CHUNK)], osems[buf])
        for co in pending_out.values():
            co.wait()

    return sc_elem


def _tc_body(sa_ref, sb_ref, x0_ref, eps_ref, out_ref):
    out_ref[...] = sa_ref[...] * x0_ref[...] + sb_ref[...] * eps_ref[...]


def _tc_apply(sa_g, sb_g, xt, et):
    off = S_ROWS // RB
    grid = ((FEAT - S_ROWS) // RB,)
    coef_spec = pl.BlockSpec((1, B), lambda i: (0, 0))
    img_spec = pl.BlockSpec((RB, B), lambda i: (i + off, 0))
    return pl.pallas_call(
        _tc_body,
        grid=grid,
        in_specs=[coef_spec, coef_spec, img_spec, img_spec],
        out_specs=img_spec,
        out_shape=jax.ShapeDtypeStruct((FEAT, B), jnp.float32),
    )(sa_g.reshape(1, B), sb_g.reshape(1, B), xt, et)


def _stitch_body(alias_ref, sc_ref, out_ref):
    out_ref[...] = sc_ref[...]


def _stitch(tc_out, sc_part):
    rb2 = 1024
    return pl.pallas_call(
        _stitch_body,
        grid=(S_ROWS // rb2,),
        in_specs=[
            pl.BlockSpec(memory_space=pltpu.MemorySpace.HBM),
            pl.BlockSpec((rb2, B), lambda i: (i, 0)),
        ],
        out_specs=pl.BlockSpec((rb2, B), lambda i: (i, 0)),
        out_shape=jax.ShapeDtypeStruct((FEAT, B), jnp.float32),
        input_output_aliases={0: 0},
    )(tc_out, sc_part)


def kernel(x0, t, eps, sqrt_alpha_bar, sqrt_one_minus_alpha_bar):
    c, h, w = x0.shape[1:]
    xt = x0.transpose(1, 2, 3, 0).reshape(FEAT, B)
    et = eps.transpose(1, 2, 3, 0).reshape(FEAT, B)
    sa_g, sb_g = _make_sc_gather()(t.astype(jnp.int32), sqrt_alpha_bar,
                                   sqrt_one_minus_alpha_bar)
    sc_part = _make_sc_elem()(xt, et, sa_g, sb_g)
    tc_out = _tc_apply(sa_g, sb_g, xt, et)
    out = _stitch(tc_out, sc_part)
    return out.reshape(c, h, w, B).transpose(3, 0, 1, 2)


# speedup vs baseline: 1.2171x; 1.2171x over previous
import functools

import jax
import jax.numpy as jnp
from jax import lax
from jax.experimental import pallas as pl
from jax.experimental.pallas import tpu as pltpu
from jax.experimental.pallas import tpu_sc as plsc

B = 1024
FEAT = 3 * 64 * 64
RB = 1024


@functools.lru_cache(maxsize=1)
def _make_sc_gather():
    info = plsc.get_sparse_core_info()
    nc, ns = info.num_cores, info.num_subcores
    nw = nc * ns
    bpw = B // nw

    mesh = plsc.VectorSubcoreMesh(core_axis_name="c", subcore_axis_name="s")

    @functools.partial(
        pl.kernel,
        mesh=mesh,
        out_type=[
            jax.ShapeDtypeStruct((B,), jnp.float32),
            jax.ShapeDtypeStruct((B,), jnp.float32),
        ],
        scratch_types=[
            pltpu.VMEM((bpw,), jnp.int32),
            pltpu.VMEM((bpw,), jnp.float32),
            pltpu.VMEM((bpw,), jnp.float32),
            pltpu.SemaphoreType.DMA,
        ],
    )
    def sc_gather(t_hbm, sa_hbm, sb_hbm, out_a_hbm, out_b_hbm, idx_v, a_v, b_v, sem):
        wid = lax.axis_index("s") * nc + lax.axis_index("c")
        base = wid * bpw
        pltpu.sync_copy(t_hbm.at[pl.ds(base, bpw)], idx_v)
        ca = pltpu.async_copy(sa_hbm.at[idx_v], a_v, sem)
        cb = pltpu.async_copy(sb_hbm.at[idx_v], b_v, sem)
        ca.wait()
        cb.wait()
        pltpu.sync_copy(a_v, out_a_hbm.at[pl.ds(base, bpw)])
        pltpu.sync_copy(b_v, out_b_hbm.at[pl.ds(base, bpw)])

    return sc_gather


def _tc_body(sa_ref, sb_ref, xa_ref, xb_ref, ea_ref, eb_ref, out_ref):
    out_ref[:RB, :] = sa_ref[...] * xa_ref[...] + sb_ref[...] * ea_ref[...]
    out_ref[RB:, :] = sa_ref[...] * xb_ref[...] + sb_ref[...] * eb_ref[...]


def _tc_apply(sa_g, sb_g, xt, et):
    grid = (FEAT // (2 * RB),)
    coef_spec = pl.BlockSpec((1, B), lambda i: (0, 0))
    in_a = pl.BlockSpec((RB, B), lambda i: (2 * i, 0))
    in_b = pl.BlockSpec((RB, B), lambda i: (2 * i + 1, 0))
    out_spec = pl.BlockSpec((2 * RB, B), lambda i: (i, 0))
    return pl.pallas_call(
        _tc_body,
        grid=grid,
        in_specs=[coef_spec, coef_spec, in_a, in_b, in_a, in_b],
        out_specs=out_spec,
        out_shape=jax.ShapeDtypeStruct((FEAT, B), jnp.float32),
    )(sa_g.reshape(1, B), sb_g.reshape(1, B), xt, xt, et, et)


def kernel(x0, t, eps, sqrt_alpha_bar, sqrt_one_minus_alpha_bar):
    c, h, w = x0.shape[1:]
    xt = x0.transpose(1, 2, 3, 0).reshape(FEAT, B)
    et = eps.transpose(1, 2, 3, 0).reshape(FEAT, B)
    sa_g, sb_g = _make_sc_gather()(t.astype(jnp.int32), sqrt_alpha_bar,
                                   sqrt_one_minus_alpha_bar)
    out = _tc_apply(sa_g, sb_g, xt, et)
    return out.reshape(c, h, w, B).transpose(3, 0, 1, 2)


# speedup vs baseline: 1.2212x; 1.0034x over previous
import functools

import jax
import jax.numpy as jnp
from jax import lax
from jax.experimental import pallas as pl
from jax.experimental.pallas import tpu as pltpu
from jax.experimental.pallas import tpu_sc as plsc

B = 1024
FEAT = 3 * 64 * 64
RB = 512


@functools.lru_cache(maxsize=1)
def _make_sc_gather():
    info = plsc.get_sparse_core_info()
    nc, ns = info.num_cores, info.num_subcores
    nw = nc * ns
    bpw = B // nw

    mesh = plsc.VectorSubcoreMesh(core_axis_name="c", subcore_axis_name="s")

    @functools.partial(
        pl.kernel,
        mesh=mesh,
        out_type=[
            jax.ShapeDtypeStruct((B,), jnp.float32),
            jax.ShapeDtypeStruct((B,), jnp.float32),
        ],
        scratch_types=[
            pltpu.VMEM((bpw,), jnp.int32),
            pltpu.VMEM((bpw,), jnp.float32),
            pltpu.VMEM((bpw,), jnp.float32),
            pltpu.SemaphoreType.DMA,
        ],
    )
    def sc_gather(t_hbm, sa_hbm, sb_hbm, out_a_hbm, out_b_hbm, idx_v, a_v, b_v, sem):
        wid = lax.axis_index("s") * nc + lax.axis_index("c")
        base = wid * bpw
        pltpu.sync_copy(t_hbm.at[pl.ds(base, bpw)], idx_v)
        ca = pltpu.async_copy(sa_hbm.at[idx_v], a_v, sem)
        cb = pltpu.async_copy(sb_hbm.at[idx_v], b_v, sem)
        ca.wait()
        cb.wait()
        pltpu.sync_copy(a_v, out_a_hbm.at[pl.ds(base, bpw)])
        pltpu.sync_copy(b_v, out_b_hbm.at[pl.ds(base, bpw)])

    return sc_gather


def _tc_body(sa_ref, sb_ref, xa_ref, xb_ref, ea_ref, eb_ref, out_ref):
    out_ref[:RB, :] = sa_ref[...] * xa_ref[...] + sb_ref[...] * ea_ref[...]
    out_ref[RB:, :] = sa_ref[...] * xb_ref[...] + sb_ref[...] * eb_ref[...]


def _tc_apply(sa_g, sb_g, xt, et):
    grid = (FEAT // (2 * RB),)
    coef_spec = pl.BlockSpec((1, B), lambda i: (0, 0))
    in_a = pl.BlockSpec((RB, B), lambda i: (2 * i, 0))
    in_b = pl.BlockSpec((RB, B), lambda i: (2 * i + 1, 0))
    out_spec = pl.BlockSpec((2 * RB, B), lambda i: (i, 0))
    return pl.pallas_call(
        _tc_body,
        grid=grid,
        in_specs=[coef_spec, coef_spec, in_a, in_b, in_a, in_b],
        out_specs=out_spec,
        out_shape=jax.ShapeDtypeStruct((FEAT, B), jnp.float32),
    )(sa_g.reshape(1, B), sb_g.reshape(1, B), xt, xt, et, et)


def kernel(x0, t, eps, sqrt_alpha_bar, sqrt_one_minus_alpha_bar):
    c, h, w = x0.shape[1:]
    xt = x0.transpose(1, 2, 3, 0).reshape(FEAT, B)
    et = eps.transpose(1, 2, 3, 0).reshape(FEAT, B)
    sa_g, sb_g = _make_sc_gather()(t.astype(jnp.int32), sqrt_alpha_bar,
                                   sqrt_one_minus_alpha_bar)
    out = _tc_apply(sa_g, sb_g, xt, et)
    return out.reshape(c, h, w, B).transpose(3, 0, 1, 2)
